# baseline (speedup 1.0000x reference)
import jax
import jax.numpy as jnp
from jax.experimental import pallas as pl
from jax.experimental.pallas import tpu as pltpu

_BB = 2


def _combine_body(t_ref, ac_ref, om_ref, x_ref, n_ref, o_ref):
    g = pl.program_id(0)
    for i in range(_BB):
        tt = t_ref[g * _BB + i]
        c1 = ac_ref[tt]
        c2 = om_ref[tt]
        o_ref[i] = c1 * x_ref[i] + c2 * n_ref[i]


def kernel(x_start, t, noise, sqrt_alphas_cumprod, sqrt_one_minus_alphas_cumprod):
    B, C, H, W = x_start.shape

    smem = pl.BlockSpec(memory_space=pltpu.SMEM)
    blk = pl.BlockSpec((_BB, C, H, W), lambda g: (g, 0, 0, 0))

    out = pl.pallas_call(
        _combine_body,
        grid=(B // _BB,),
        in_specs=[smem, smem, smem, blk, blk],
        out_specs=blk,
        out_shape=jax.ShapeDtypeStruct((B, C, H, W), jnp.float32),
    )(t.astype(jnp.int32), sqrt_alphas_cumprod, sqrt_one_minus_alphas_cumprod,
      x_start, noise)
    return out
